# baseline (speedup 1.0000x reference)
import functools

import jax
import jax.numpy as jnp
from jax import lax
from jax.experimental import pallas as pl
from jax.experimental.pallas import tpu as pltpu
from jax.experimental.pallas import tpu_sc as plsc

NC = 2
NS = 16
NW = NC * NS


def kernel(x, class_idx, gamma, beta):
    B, H, W, C = x.shape
    idx = class_idx.astype(jnp.int32)
    rows_per_worker = B * H // NW
    workers_per_batch = H // rows_per_worker
    n_groups = C // 16

    @functools.partial(
        pl.kernel,
        out_type=jax.ShapeDtypeStruct((B, H, W, C), jnp.float32),
        mesh=plsc.VectorSubcoreMesh(core_axis_name="c", subcore_axis_name="s"),
        scratch_types=[
            pltpu.VMEM((16,), jnp.int32),
            pltpu.VMEM((C,), jnp.float32),
            pltpu.VMEM((C,), jnp.float32),
            pltpu.VMEM((W, C), jnp.float32),
            pltpu.VMEM((W, C), jnp.float32),
            pltpu.VMEM((W, C), jnp.float32),
            pltpu.VMEM((W, C), jnp.float32),
            pltpu.SemaphoreType.DMA,
            pltpu.SemaphoreType.DMA,
            pltpu.SemaphoreType.DMA,
            pltpu.SemaphoreType.DMA,
        ],
    )
    def sc_affine(x_hbm, idx_hbm, gamma_hbm, beta_hbm, o_hbm,
                  idx_v, grow, trow, in0, in1, ou0, ou1,
                  isem0, isem1, osem0, osem1):
        cid = lax.axis_index("c")
        sid = lax.axis_index("s")
        wid = sid * NC + cid
        b = wid // workers_per_batch
        h_base = (wid % workers_per_batch) * rows_per_worker

        pltpu.sync_copy(idx_hbm, idx_v.at[pl.ds(0, B)])
        iv = idx_v[...]
        cls = jnp.int32(0)
        for bb in range(B):
            cls = jnp.where(b == bb, iv[bb], cls)
        pltpu.sync_copy(gamma_hbm.at[cls], grow)
        pltpu.sync_copy(beta_hbm.at[cls], trow)
        gv = [grow[pl.ds(k * 16, 16)] for k in range(n_groups)]
        tv = [trow[pl.ds(k * 16, 16)] for k in range(n_groups)]

        inbuf = (in0, in1)
        oubuf = (ou0, ou1)
        isem = (isem0, isem1)
        osem = (osem0, osem1)

        def in_copy(j, s):
            return pltpu.make_async_copy(
                x_hbm.at[b, h_base + j], inbuf[s], isem[s])

        def out_copy(j, s):
            return pltpu.make_async_copy(
                oubuf[s], o_hbm.at[b, h_base + j], osem[s])

        in_copy(0, 0).start()
        in_copy(1, 1).start()

        def compute(s):
            def wstep(w, carry):
                for k in range(n_groups):
                    sl = pl.ds(k * 16, 16)
                    oubuf[s][w, sl] = inbuf[s][w, sl] * gv[k] + tv[k]
                return carry
            lax.fori_loop(0, W, wstep, 0)

        def round_step(r, carry):
            for s in range(2):
                j = 2 * r + s
                in_copy(j, s).wait()

                @pl.when(r >= 1)
                def _():
                    out_copy(j - 2, s).wait()

                compute(s)
                out_copy(j, s).start()

                @pl.when(j + 2 < rows_per_worker)
                def _():
                    in_copy(j + 2, s).start()
            return carry

        lax.fori_loop(0, rows_per_worker // 2, round_step, 0)

        out_copy(rows_per_worker - 2, 0).wait()
        out_copy(rows_per_worker - 1, 1).wait()

    return sc_affine(x, idx, gamma, beta)


# speedup vs baseline: 1.0368x; 1.0368x over previous
import functools

import jax
import jax.numpy as jnp
from jax import lax
from jax.experimental import pallas as pl
from jax.experimental.pallas import tpu as pltpu
from jax.experimental.pallas import tpu_sc as plsc


def _gather_params_sc(gamma, beta, idx, B, C):
    @functools.partial(
        pl.kernel,
        out_type=(
            jax.ShapeDtypeStruct((B, C), jnp.float32),
            jax.ShapeDtypeStruct((B, C), jnp.float32),
        ),
        mesh=plsc.VectorSubcoreMesh(core_axis_name="c", subcore_axis_name="s"),
        scratch_types=[
            pltpu.VMEM((16,), jnp.int32),
            pltpu.VMEM((B, C), jnp.float32),
        ],
    )
    def gather_kernel(gamma_hbm, beta_hbm, idx_hbm, g_out, t_out, idx_v, rows_v):
        cid = lax.axis_index("c")
        sid = lax.axis_index("s")

        @pl.when(jnp.logical_and(cid == 0, sid == 0))
        def _():
            pltpu.sync_copy(idx_hbm, idx_v.at[pl.ds(0, B)])
            iv = idx_v[...]
            for b in range(B):
                pltpu.sync_copy(gamma_hbm.at[iv[b]], rows_v.at[b])
            pltpu.sync_copy(rows_v, g_out)

        @pl.when(jnp.logical_and(cid == 1, sid == 0))
        def _():
            pltpu.sync_copy(idx_hbm, idx_v.at[pl.ds(0, B)])
            iv = idx_v[...]
            for b in range(B):
                pltpu.sync_copy(beta_hbm.at[iv[b]], rows_v.at[b])
            pltpu.sync_copy(rows_v, t_out)

    return gather_kernel(gamma, beta, idx)


def _affine_body(x_ref, g_ref, t_ref, o_ref):
    b = pl.program_id(0)
    g = g_ref[pl.ds(b, 1), :]
    t = t_ref[pl.ds(b, 1), :]
    o_ref[...] = x_ref[...] * g[0][None, None, None, :] + t[0][None, None, None, :]


def kernel(x, class_idx, gamma, beta):
    B, H, W, C = x.shape
    idx = class_idx.astype(jnp.int32)

    g_sel, t_sel = _gather_params_sc(gamma, beta, idx, B, C)

    TH = 112
    assert H % TH == 0
    out = pl.pallas_call(
        _affine_body,
        grid=(B, H // TH),
        in_specs=[
            pl.BlockSpec((1, TH, W, C), lambda b, h: (b, h, 0, 0)),
            pl.BlockSpec((B, C), lambda b, h: (0, 0)),
            pl.BlockSpec((B, C), lambda b, h: (0, 0)),
        ],
        out_specs=pl.BlockSpec((1, TH, W, C), lambda b, h: (b, h, 0, 0)),
        out_shape=jax.ShapeDtypeStruct((B, H, W, C), jnp.float32),
        compiler_params=pltpu.CompilerParams(
            dimension_semantics=("parallel", "arbitrary"),
        ),
    )(x, g_sel, t_sel)

    return out
